# Initial kernel scaffold; baseline (speedup 1.0000x reference)
import jax
import jax.numpy as jnp
from jax.experimental import pallas as pl


def kernel(x, pos_table):
    raise NotImplementedError("write your pallas kernel here")


# speedup vs baseline: 1.5006x; 1.5006x over previous
import jax
import jax.numpy as jnp
from jax.experimental import pallas as pl


def _body(x_ref, p_ref, o_ref):
    o_ref[...] = x_ref[...] + p_ref[...]


def kernel(x, pos_table):
    b, s, d = x.shape
    sb = 512
    grid = (s // sb, b)
    return pl.pallas_call(
        _body,
        grid=grid,
        in_specs=[
            pl.BlockSpec((1, sb, d), lambda i, j: (j, i, 0)),
            pl.BlockSpec((sb, d), lambda i, j: (i, 0)),
        ],
        out_specs=pl.BlockSpec((1, sb, d), lambda i, j: (j, i, 0)),
        out_shape=jax.ShapeDtypeStruct((b, s, d), x.dtype),
    )(x, pos_table)


# speedup vs baseline: 1.6687x; 1.1120x over previous
import jax
import jax.numpy as jnp
from jax.experimental import pallas as pl


def _body(x_ref, p_ref, o_ref):
    o_ref[...] = x_ref[...] + p_ref[...]


def kernel(x, pos_table):
    b, s, d = x.shape
    sb = 1024
    grid = (s // sb, b)
    return pl.pallas_call(
        _body,
        grid=grid,
        in_specs=[
            pl.BlockSpec((1, sb, d), lambda i, j: (j, i, 0)),
            pl.BlockSpec((sb, d), lambda i, j: (i, 0)),
        ],
        out_specs=pl.BlockSpec((1, sb, d), lambda i, j: (j, i, 0)),
        out_shape=jax.ShapeDtypeStruct((b, s, d), x.dtype),
    )(x, pos_table)


# speedup vs baseline: 1.7326x; 1.0383x over previous
import jax
import jax.numpy as jnp
from jax.experimental import pallas as pl


def _body(x_ref, p_ref, o_ref):
    o_ref[...] = x_ref[...] + p_ref[...]


def kernel(x, pos_table):
    b, s, d = x.shape
    sb = 2048
    grid = (s // sb, b)
    return pl.pallas_call(
        _body,
        grid=grid,
        in_specs=[
            pl.BlockSpec((1, sb, d), lambda i, j: (j, i, 0)),
            pl.BlockSpec((sb, d), lambda i, j: (i, 0)),
        ],
        out_specs=pl.BlockSpec((1, sb, d), lambda i, j: (j, i, 0)),
        out_shape=jax.ShapeDtypeStruct((b, s, d), x.dtype),
    )(x, pos_table)


# speedup vs baseline: 1.7359x; 1.0019x over previous
import jax
import jax.numpy as jnp
from jax.experimental import pallas as pl
from jax.experimental.pallas import tpu as pltpu


def _body(x_ref, p_ref, o_ref):
    o_ref[...] = x_ref[...] + p_ref[...]


def kernel(x, pos_table):
    b, s, d = x.shape
    sb = 2048
    grid = (s // sb, b)
    return pl.pallas_call(
        _body,
        grid=grid,
        in_specs=[
            pl.BlockSpec((1, sb, d), lambda i, j: (j, i, 0)),
            pl.BlockSpec((sb, d), lambda i, j: (i, 0)),
        ],
        out_specs=pl.BlockSpec((1, sb, d), lambda i, j: (j, i, 0)),
        out_shape=jax.ShapeDtypeStruct((b, s, d), x.dtype),
        compiler_params=pltpu.CompilerParams(
            dimension_semantics=("parallel", "parallel"),
        ),
    )(x, pos_table)
